# baseline (speedup 1.0000x reference)
import functools

import jax
import jax.numpy as jnp
from jax import lax
from jax.experimental import pallas as pl
from jax.experimental.pallas import tpu as pltpu
from jax.experimental.pallas import tpu_sc as plsc

_D_MODEL = 4096
_NUM_EXPERTS = 64
_TOP_K = 8
_BLOCK_T = 1024
_N_TOKENS = 32768

_NW = 32
_ROWS_PER_W = _N_TOKENS // _NW
_CHUNK = 128
_GROUPS = _CHUNK // 16


def _gate_tc_kernel(x_ref, w_ref, probs_ref, logits_ref, probst_ref):
    logits_t = jax.lax.dot_general(
        w_ref[...], x_ref[...],
        dimension_numbers=(((1,), (1,)), ((), ())),
        preferred_element_type=jnp.float32,
    )
    logits_ref[...] = logits_t.T
    m = jnp.max(logits_t, axis=0, keepdims=True)
    e = jnp.exp(logits_t - m)
    s = jnp.sum(e, axis=0, keepdims=True)
    probs_t = e / s
    probst_ref[...] = probs_t
    probs_ref[...] = probs_t.T


def _sc_topk_body(probst_hbm, outwt_hbm, outit_hbm, buf, ow, oi):
    c_idx = lax.axis_index("c")
    s_idx = lax.axis_index("s")
    wid = s_idx * 2 + c_idx
    base = wid * _ROWS_PER_W

    def do_chunk(c, carry0):
        t0 = base + c * _CHUNK
        pltpu.sync_copy(probst_hbm.at[:, pl.ds(t0, _CHUNK)], buf)

        def do_group(g, carry1):

            def do_expert(e_i, carry):
                ws, idxs, psum = carry
                v = buf[e_i, pl.ds(g * 16, 16)]
                psum = psum + v
                vi = jnp.full((16,), e_i, jnp.int32)
                new_ws = []
                new_is = []
                for k in range(_TOP_K):
                    gt = v > ws[k]
                    new_ws.append(jnp.where(gt, v, ws[k]))
                    new_is.append(jnp.where(gt, vi, idxs[k]))
                    v = jnp.where(gt, ws[k], v)
                    vi = jnp.where(gt, idxs[k], vi)
                return (tuple(new_ws), tuple(new_is), psum)

            neg = jnp.full((16,), -1.0, jnp.float32)
            zero_i = jnp.full((16,), 0, jnp.int32)
            ws, idxs, psum = lax.fori_loop(
                0, _NUM_EXPERTS, do_expert,
                ((neg,) * _TOP_K, (zero_i,) * _TOP_K,
                 jnp.full((16,), 0.0, jnp.float32)),
            )
            rcp = 1.0 / psum
            for k in range(_TOP_K):
                ow[k, pl.ds(g * 16, 16)] = ws[k] * rcp
                oi[k, pl.ds(g * 16, 16)] = idxs[k]
            return carry1

        lax.fori_loop(0, _GROUPS, do_group, 0)
        pltpu.sync_copy(ow, outwt_hbm.at[:, pl.ds(t0, _CHUNK)])
        pltpu.sync_copy(oi, outit_hbm.at[:, pl.ds(t0, _CHUNK)])
        return carry0

    lax.fori_loop(0, _ROWS_PER_W // _CHUNK, do_chunk, 0)


_sc_topk = functools.partial(
    pl.kernel,
    out_type=[
        jax.ShapeDtypeStruct((_TOP_K, _N_TOKENS), jnp.float32),
        jax.ShapeDtypeStruct((_TOP_K, _N_TOKENS), jnp.int32),
    ],
    mesh=plsc.VectorSubcoreMesh(core_axis_name="c", subcore_axis_name="s"),
    scratch_types=[
        pltpu.VMEM((_NUM_EXPERTS, _CHUNK), jnp.float32),
        pltpu.VMEM((_TOP_K, _CHUNK), jnp.float32),
        pltpu.VMEM((_TOP_K, _CHUNK), jnp.int32),
    ],
)(_sc_topk_body)


@functools.partial(jax.jit, static_argnames=())
def kernel(x, W):
    n_tokens, d_model = x.shape
    n_experts = W.shape[0]
    grid = (n_tokens // _BLOCK_T,)
    probs, logits, probs_t = pl.pallas_call(
        _gate_tc_kernel,
        grid=grid,
        in_specs=[
            pl.BlockSpec((_BLOCK_T, d_model), lambda i: (i, 0)),
            pl.BlockSpec((n_experts, d_model), lambda i: (0, 0)),
        ],
        out_specs=[
            pl.BlockSpec((_BLOCK_T, n_experts), lambda i: (i, 0)),
            pl.BlockSpec((_BLOCK_T, n_experts), lambda i: (i, 0)),
            pl.BlockSpec((n_experts, _BLOCK_T), lambda i: (0, i)),
        ],
        out_shape=[
            jax.ShapeDtypeStruct((n_tokens, n_experts), jnp.float32),
            jax.ShapeDtypeStruct((n_tokens, n_experts), jnp.float32),
            jax.ShapeDtypeStruct((n_experts, n_tokens), jnp.float32),
        ],
        compiler_params=pltpu.CompilerParams(
            dimension_semantics=("arbitrary",),
        ),
    )(x, W)
    topk_w_t, topk_i_t = _sc_topk(probs_t)
    return (topk_w_t.T, probs, topk_i_t.T, logits)
